# baseline (speedup 1.0000x reference)
import jax
import jax.numpy as jnp
from jax import lax
from jax.experimental import pallas as pl
from jax.experimental.pallas import tpu as pltpu
from jax.experimental.pallas import tpu_sc as plsc

N = 10000
E = 320000
F = 128
H = 128
G = 64

NC = 2
NS = 16
NW = NC * NS
CHUNK = 128
CT = 2 * (-(-E // (NW * CHUNK * 2)))
EPT = CT * CHUNK
EPAD = NW * EPT
NP = N + 8
DEGW = 128

ROWS_T = 10
ROWS = N // ROWS_T


def _sc_mesh():
    return plsc.VectorSubcoreMesh(core_axis_name="c", subcore_axis_name="s")


def _unpack_src(packed_v, j, sidx, b):
    for k in range(CHUNK // 16):
        v = packed_v.at[j][pl.ds(16 * k, 16)]
        sidx.at[b][pl.ds(16 * k, 16)] = lax.bitwise_and(v, jnp.int32(0x3FFF))


def _unpack_dst(packed_v, j, didx, b):
    for k in range(CHUNK // 16):
        v = packed_v.at[j][pl.ds(16 * k, 16)]
        didx.at[b][pl.ds(16 * k, 16)] = lax.shift_right_logical(
            v, jnp.int32(14))


def _deg_body(pk_hbm, zeros_hbm, ones_hbm, out_hbm, pk_v, didx, ones_v, acc_sh):
    c = lax.axis_index("c")
    s = lax.axis_index("s")
    t = c * NS + s
    pltpu.sync_copy(pk_hbm.at[t], pk_v)
    pltpu.sync_copy(ones_hbm, ones_v)

    @pl.when(s == 0)
    def _():
        pltpu.sync_copy(zeros_hbm, acc_sh)

    plsc.subcore_barrier()

    def chunk(j, carry):
        _unpack_dst(pk_v, j, didx, 0)
        pltpu.sync_copy(ones_v, acc_sh.at[didx.at[0]], add=True)
        return carry

    lax.fori_loop(0, CT, chunk, 0)
    plsc.subcore_barrier()

    @pl.when(s == 0)
    def _():
        pltpu.sync_copy(acc_sh, out_hbm.at[c])


def _sc_degree(packed_t, zeros16, ones16):
    k = pl.kernel(
        _deg_body,
        out_type=jax.ShapeDtypeStruct((NC, NP, DEGW), jnp.float32),
        mesh=_sc_mesh(),
        scratch_types=[
            pltpu.VMEM((CT, CHUNK), jnp.int32),
            pltpu.VMEM((1, CHUNK), jnp.int32),
            pltpu.VMEM((CHUNK, DEGW), jnp.float32),
            pltpu.VMEM_SHARED((NP, DEGW), jnp.float32),
        ],
    )
    return k(packed_t, zeros16, ones16)


def _agg_body(g_hbm, pk_hbm, zeros_hbm, out_hbm,
              pk_v, sidx, didx, rows_v, gsem, acc_sh):
    c = lax.axis_index("c")
    s = lax.axis_index("s")
    t = c * NS + s
    pltpu.sync_copy(pk_hbm.at[t], pk_v)

    @pl.when(s == 0)
    def _():
        pltpu.sync_copy(zeros_hbm, acc_sh)

    plsc.subcore_barrier()

    def pair(g2, carry):
        j0 = 2 * g2
        j1 = j0 + 1
        _unpack_src(pk_v, j0, sidx, 0)
        _unpack_dst(pk_v, j0, didx, 0)
        _unpack_src(pk_v, j1, sidx, 1)
        _unpack_dst(pk_v, j1, didx, 1)
        d0 = pltpu.async_copy(g_hbm.at[sidx.at[0]], rows_v.at[0], gsem)
        d1 = pltpu.async_copy(g_hbm.at[sidx.at[1]], rows_v.at[1], gsem)
        d0.wait()
        d1.wait()
        pltpu.sync_copy(rows_v.at[0], acc_sh.at[didx.at[0]], add=True)
        pltpu.sync_copy(rows_v.at[1], acc_sh.at[didx.at[1]], add=True)
        return carry

    lax.fori_loop(0, CT // 2, pair, 0)
    plsc.subcore_barrier()

    @pl.when(s == 0)
    def _():
        pltpu.sync_copy(acc_sh, out_hbm.at[c])


def _sc_aggregate(g, packed_t, zeros_np):
    k = pl.kernel(
        _agg_body,
        out_type=jax.ShapeDtypeStruct((NC, NP, H), jnp.float32),
        mesh=_sc_mesh(),
        scratch_types=[
            pltpu.VMEM((CT, CHUNK), jnp.int32),
            pltpu.VMEM((2, CHUNK), jnp.int32),
            pltpu.VMEM((2, CHUNK), jnp.int32),
            pltpu.VMEM((2, CHUNK, H), jnp.float32),
            pltpu.SemaphoreType.DMA,
            pltpu.VMEM_SHARED((NP, H), jnp.float32),
        ],
    )
    return k(g, packed_t, zeros_np)


def _mm_scale_body(x_ref, w_ref, deg_ref, g_ref, dinv_ref):
    t = jnp.dot(x_ref[...], w_ref[...], preferred_element_type=jnp.float32)
    dinv = lax.rsqrt(deg_ref[...] + 1.0)
    dinv_ref[...] = dinv
    g_ref[...] = t * dinv


def _tc_mm_scale(x, w, deg_e):
    return pl.pallas_call(
        _mm_scale_body,
        grid=(ROWS_T,),
        in_specs=[
            pl.BlockSpec((ROWS, F), lambda i: (i, 0)),
            pl.BlockSpec((F, H), lambda i: (0, 0)),
            pl.BlockSpec((ROWS, 1), lambda i: (i, 0)),
        ],
        out_specs=[
            pl.BlockSpec((ROWS, H), lambda i: (i, 0)),
            pl.BlockSpec((ROWS, 1), lambda i: (i, 0)),
        ],
        out_shape=[
            jax.ShapeDtypeStruct((N, H), jnp.float32),
            jax.ShapeDtypeStruct((N, 1), jnp.float32),
        ],
    )(x, w, deg_e)


def _mid_body(a0_ref, a1_ref, g_ref, dinv_ref, b_ref, w_ref, out_ref):
    dinv = dinv_ref[...]
    h = (a0_ref[...] + a1_ref[...] + g_ref[...]) * dinv + b_ref[...]
    h = jnp.maximum(h, 0.0)
    t = jnp.dot(h, w_ref[...], preferred_element_type=jnp.float32)
    out_ref[...] = t * dinv


def _tc_mid(a0, a1, g, dinv, b, w):
    return pl.pallas_call(
        _mid_body,
        grid=(ROWS_T,),
        in_specs=[
            pl.BlockSpec((ROWS, H), lambda i: (i, 0)),
            pl.BlockSpec((ROWS, H), lambda i: (i, 0)),
            pl.BlockSpec((ROWS, H), lambda i: (i, 0)),
            pl.BlockSpec((ROWS, 1), lambda i: (i, 0)),
            pl.BlockSpec((1, H), lambda i: (0, 0)),
            pl.BlockSpec((H, H), lambda i: (0, 0)),
        ],
        out_specs=pl.BlockSpec((ROWS, H), lambda i: (i, 0)),
        out_shape=jax.ShapeDtypeStruct((N, H), jnp.float32),
    )(a0, a1, g, dinv, b, w)


def _pool_body(a0_ref, a1_ref, g_ref, dinv_ref, b_ref, batch_ref,
               wl_ref, bl_ref, out_ref, psum, pcnt):
    i = pl.program_id(0)

    @pl.when(i == 0)
    def _():
        psum[...] = jnp.zeros_like(psum)
        pcnt[...] = jnp.zeros_like(pcnt)

    h = (a0_ref[...] + a1_ref[...] + g_ref[...]) * dinv_ref[...] + b_ref[...]
    ids = lax.broadcasted_iota(jnp.int32, (ROWS, G), 1)
    oh = (batch_ref[...] == ids).astype(jnp.float32)
    psum[...] += lax.dot_general(oh, h, (((0,), (0,)), ((), ())),
                                 preferred_element_type=jnp.float32)
    pcnt[...] += jnp.sum(oh, axis=0)[:, None]

    @pl.when(i == ROWS_T - 1)
    def _():
        pooled = psum[...] / jnp.maximum(pcnt[...], 1.0)
        out_ref[...] = jnp.dot(pooled, wl_ref[...],
                               preferred_element_type=jnp.float32) + bl_ref[...]


def _tc_pool(a0, a1, g, dinv, b, batch2d, wl, bl):
    return pl.pallas_call(
        _pool_body,
        grid=(ROWS_T,),
        in_specs=[
            pl.BlockSpec((ROWS, H), lambda i: (i, 0)),
            pl.BlockSpec((ROWS, H), lambda i: (i, 0)),
            pl.BlockSpec((ROWS, H), lambda i: (i, 0)),
            pl.BlockSpec((ROWS, 1), lambda i: (i, 0)),
            pl.BlockSpec((1, H), lambda i: (0, 0)),
            pl.BlockSpec((ROWS, 1), lambda i: (i, 0)),
            pl.BlockSpec((H, 2), lambda i: (0, 0)),
            pl.BlockSpec((1, 2), lambda i: (0, 0)),
        ],
        out_specs=pl.BlockSpec((G, 2), lambda i: (0, 0)),
        out_shape=jax.ShapeDtypeStruct((G, 2), jnp.float32),
        scratch_shapes=[
            pltpu.VMEM((G, H), jnp.float32),
            pltpu.VMEM((G, 1), jnp.float32),
        ],
    )(a0, a1, g, dinv, b, batch2d, wl, bl)


def kernel(x, edge_index, batch, W1, b1, W2, b2, Wl, bl):
    src = edge_index[0]
    dst = edge_index[1]
    pad = EPAD - E
    packed = src + dst * 16384
    packed_t = jnp.concatenate(
        [packed, jnp.full((pad,), N * 16384, jnp.int32)]).reshape(NW, CT, CHUNK)

    zeros_np = jnp.zeros((NP, H), jnp.float32)
    ones_ch = jnp.ones((CHUNK, DEGW), jnp.float32)
    batch2d = batch.astype(jnp.int32)[:, None]

    degp = _sc_degree(packed_t, zeros_np, ones_ch)
    deg_e = degp[0, :N, :1] + degp[1, :N, :1]

    g1, dinv = _tc_mm_scale(x, W1, deg_e)
    acc1 = _sc_aggregate(g1, packed_t, zeros_np)
    g2 = _tc_mid(acc1[0, :N], acc1[1, :N], g1, dinv, b1[None, :], W2)
    acc2 = _sc_aggregate(g2, packed_t, zeros_np)
    out = _tc_pool(acc2[0, :N], acc2[1, :N], g2, dinv, b2[None, :],
                   batch2d, Wl, bl[None, :])
    return out


# speedup vs baseline: 1.0703x; 1.0703x over previous
import jax
import jax.numpy as jnp
from jax import lax
from jax.experimental import pallas as pl
from jax.experimental.pallas import tpu as pltpu
from jax.experimental.pallas import tpu_sc as plsc

N = 10000
E = 320000
F = 128
H = 128
G = 64

NC = 2
NS = 16
NW = NC * NS
CHUNK = 128
CT = 2 * (-(-E // (NW * CHUNK * 2)))
EPT = CT * CHUNK
EPAD = NW * EPT
NP = N + 8
DEGW = 128

ROWS_T = 10
ROWS = N // ROWS_T


def _sc_mesh():
    return plsc.VectorSubcoreMesh(core_axis_name="c", subcore_axis_name="s")


def _unpack_src(packed_v, j, sidx, b):
    for k in range(CHUNK // 16):
        v = packed_v.at[j][pl.ds(16 * k, 16)]
        sidx.at[b][pl.ds(16 * k, 16)] = lax.bitwise_and(v, jnp.int32(0x3FFF))


def _unpack_dst(packed_v, j, didx, b):
    for k in range(CHUNK // 16):
        v = packed_v.at[j][pl.ds(16 * k, 16)]
        didx.at[b][pl.ds(16 * k, 16)] = lax.shift_right_logical(
            v, jnp.int32(14))


def _deg_body(pk_hbm, zeros_hbm, ones_hbm, out_hbm, pk_v, didx, ones_v, ssem,
              acc_sh):
    c = lax.axis_index("c")
    s = lax.axis_index("s")
    t = c * NS + s
    pltpu.sync_copy(pk_hbm.at[t], pk_v)
    pltpu.sync_copy(ones_hbm, ones_v)

    @pl.when(s == 0)
    def _():
        pltpu.sync_copy(zeros_hbm, acc_sh)

    plsc.subcore_barrier()

    def pair(g2, carry):
        j0 = 2 * g2
        j1 = j0 + 1
        _unpack_dst(pk_v, j0, didx, 0)
        _unpack_dst(pk_v, j1, didx, 1)
        d0 = pltpu.async_copy(ones_v, acc_sh.at[didx.at[0]], ssem, add=True)
        d1 = pltpu.async_copy(ones_v, acc_sh.at[didx.at[1]], ssem, add=True)
        d0.wait()
        d1.wait()
        return carry

    lax.fori_loop(0, CT // 2, pair, 0)
    plsc.subcore_barrier()

    @pl.when(s == 0)
    def _():
        pltpu.sync_copy(acc_sh, out_hbm.at[c])


def _sc_degree(packed_t, zeros128, ones128):
    k = pl.kernel(
        _deg_body,
        out_type=jax.ShapeDtypeStruct((NC, NP, DEGW), jnp.float32),
        mesh=_sc_mesh(),
        scratch_types=[
            pltpu.VMEM((CT, CHUNK), jnp.int32),
            pltpu.VMEM((2, CHUNK), jnp.int32),
            pltpu.VMEM((CHUNK, DEGW), jnp.float32),
            pltpu.SemaphoreType.DMA,
            pltpu.VMEM_SHARED((NP, DEGW), jnp.float32),
        ],
    )
    return k(packed_t, zeros128, ones128)


def _agg_body(g_hbm, pk_hbm, zeros_hbm, out_hbm,
              pk_v, sidx, didx, rows_v, gsem, acc_sh):
    c = lax.axis_index("c")
    s = lax.axis_index("s")
    t = c * NS + s
    pltpu.sync_copy(pk_hbm.at[t], pk_v)

    @pl.when(s == 0)
    def _():
        pltpu.sync_copy(zeros_hbm, acc_sh)

    plsc.subcore_barrier()

    _unpack_src(pk_v, 0, sidx, 0)
    _unpack_dst(pk_v, 0, didx, 0)
    pltpu.async_copy(g_hbm.at[sidx.at[0]], rows_v.at[0], gsem)

    def pair(g2, carry):
        j0 = 2 * g2
        j1 = j0 + 1
        _unpack_src(pk_v, j1, sidx, 1)
        _unpack_dst(pk_v, j1, didx, 1)
        pltpu.make_async_copy(g_hbm.at[sidx.at[0]], rows_v.at[0], gsem).wait()
        pltpu.async_copy(g_hbm.at[sidx.at[1]], rows_v.at[1], gsem)
        pltpu.sync_copy(rows_v.at[0], acc_sh.at[didx.at[0]], add=True)

        @pl.when(j1 + 1 < CT)
        def _():
            _unpack_src(pk_v, j1 + 1, sidx, 0)
            _unpack_dst(pk_v, j1 + 1, didx, 0)

        pltpu.make_async_copy(g_hbm.at[sidx.at[1]], rows_v.at[1], gsem).wait()

        @pl.when(j1 + 1 < CT)
        def _():
            pltpu.async_copy(g_hbm.at[sidx.at[0]], rows_v.at[0], gsem)

        pltpu.sync_copy(rows_v.at[1], acc_sh.at[didx.at[1]], add=True)
        return carry

    lax.fori_loop(0, CT // 2, pair, 0)
    plsc.subcore_barrier()

    @pl.when(s == 0)
    def _():
        pltpu.sync_copy(acc_sh, out_hbm.at[c])


def _sc_aggregate(g, packed_t, zeros_np):
    k = pl.kernel(
        _agg_body,
        out_type=jax.ShapeDtypeStruct((NC, NP, H), jnp.float32),
        mesh=_sc_mesh(),
        scratch_types=[
            pltpu.VMEM((CT, CHUNK), jnp.int32),
            pltpu.VMEM((2, CHUNK), jnp.int32),
            pltpu.VMEM((2, CHUNK), jnp.int32),
            pltpu.VMEM((2, CHUNK, H), jnp.float32),
            pltpu.SemaphoreType.DMA,
            pltpu.VMEM_SHARED((NP, H), jnp.float32),
        ],
    )
    return k(g, packed_t, zeros_np)


def _mm_scale_body(x_ref, w_ref, deg_ref, g_ref, dinv_ref):
    t = jnp.dot(x_ref[...], w_ref[...], preferred_element_type=jnp.float32)
    dinv = lax.rsqrt(deg_ref[...] + 1.0)
    dinv_ref[...] = dinv
    g_ref[...] = t * dinv


def _tc_mm_scale(x, w, deg_e):
    return pl.pallas_call(
        _mm_scale_body,
        grid=(ROWS_T,),
        in_specs=[
            pl.BlockSpec((ROWS, F), lambda i: (i, 0)),
            pl.BlockSpec((F, H), lambda i: (0, 0)),
            pl.BlockSpec((ROWS, 1), lambda i: (i, 0)),
        ],
        out_specs=[
            pl.BlockSpec((ROWS, H), lambda i: (i, 0)),
            pl.BlockSpec((ROWS, 1), lambda i: (i, 0)),
        ],
        out_shape=[
            jax.ShapeDtypeStruct((N, H), jnp.float32),
            jax.ShapeDtypeStruct((N, 1), jnp.float32),
        ],
    )(x, w, deg_e)


def _mid_body(a0_ref, a1_ref, g_ref, dinv_ref, b_ref, w_ref, out_ref):
    dinv = dinv_ref[...]
    h = (a0_ref[...] + a1_ref[...] + g_ref[...]) * dinv + b_ref[...]
    h = jnp.maximum(h, 0.0)
    t = jnp.dot(h, w_ref[...], preferred_element_type=jnp.float32)
    out_ref[...] = t * dinv


def _tc_mid(a0, a1, g, dinv, b, w):
    return pl.pallas_call(
        _mid_body,
        grid=(ROWS_T,),
        in_specs=[
            pl.BlockSpec((ROWS, H), lambda i: (i, 0)),
            pl.BlockSpec((ROWS, H), lambda i: (i, 0)),
            pl.BlockSpec((ROWS, H), lambda i: (i, 0)),
            pl.BlockSpec((ROWS, 1), lambda i: (i, 0)),
            pl.BlockSpec((1, H), lambda i: (0, 0)),
            pl.BlockSpec((H, H), lambda i: (0, 0)),
        ],
        out_specs=pl.BlockSpec((ROWS, H), lambda i: (i, 0)),
        out_shape=jax.ShapeDtypeStruct((N, H), jnp.float32),
    )(a0, a1, g, dinv, b, w)


def _pool_body(a0_ref, a1_ref, g_ref, dinv_ref, b_ref, batch_ref,
               wl_ref, bl_ref, out_ref, psum, pcnt):
    i = pl.program_id(0)

    @pl.when(i == 0)
    def _():
        psum[...] = jnp.zeros_like(psum)
        pcnt[...] = jnp.zeros_like(pcnt)

    h = (a0_ref[...] + a1_ref[...] + g_ref[...]) * dinv_ref[...] + b_ref[...]
    ids = lax.broadcasted_iota(jnp.int32, (ROWS, G), 1)
    oh = (batch_ref[...] == ids).astype(jnp.float32)
    psum[...] += lax.dot_general(oh, h, (((0,), (0,)), ((), ())),
                                 preferred_element_type=jnp.float32)
    pcnt[...] += jnp.sum(oh, axis=0)[:, None]

    @pl.when(i == ROWS_T - 1)
    def _():
        pooled = psum[...] / jnp.maximum(pcnt[...], 1.0)
        out_ref[...] = jnp.dot(pooled, wl_ref[...],
                               preferred_element_type=jnp.float32) + bl_ref[...]


def _tc_pool(a0, a1, g, dinv, b, batch2d, wl, bl):
    return pl.pallas_call(
        _pool_body,
        grid=(ROWS_T,),
        in_specs=[
            pl.BlockSpec((ROWS, H), lambda i: (i, 0)),
            pl.BlockSpec((ROWS, H), lambda i: (i, 0)),
            pl.BlockSpec((ROWS, H), lambda i: (i, 0)),
            pl.BlockSpec((ROWS, 1), lambda i: (i, 0)),
            pl.BlockSpec((1, H), lambda i: (0, 0)),
            pl.BlockSpec((ROWS, 1), lambda i: (i, 0)),
            pl.BlockSpec((H, 2), lambda i: (0, 0)),
            pl.BlockSpec((1, 2), lambda i: (0, 0)),
        ],
        out_specs=pl.BlockSpec((G, 2), lambda i: (0, 0)),
        out_shape=jax.ShapeDtypeStruct((G, 2), jnp.float32),
        scratch_shapes=[
            pltpu.VMEM((G, H), jnp.float32),
            pltpu.VMEM((G, 1), jnp.float32),
        ],
    )(a0, a1, g, dinv, b, batch2d, wl, bl)


def kernel(x, edge_index, batch, W1, b1, W2, b2, Wl, bl):
    src = edge_index[0]
    dst = edge_index[1]
    pad = EPAD - E
    packed = src + dst * 16384
    packed_t = jnp.concatenate(
        [packed, jnp.full((pad,), N * 16384, jnp.int32)]).reshape(NW, CT, CHUNK)

    zeros_np = jnp.zeros((NP, H), jnp.float32)
    ones_ch = jnp.ones((CHUNK, DEGW), jnp.float32)
    batch2d = batch.astype(jnp.int32)[:, None]

    degp = _sc_degree(packed_t, zeros_np, ones_ch)
    deg_e = degp[0, :N, :1] + degp[1, :N, :1]

    g1, dinv = _tc_mm_scale(x, W1, deg_e)
    acc1 = _sc_aggregate(g1, packed_t, zeros_np)
    g2 = _tc_mid(acc1[0, :N], acc1[1, :N], g1, dinv, b1[None, :], W2)
    acc2 = _sc_aggregate(g2, packed_t, zeros_np)
    out = _tc_pool(acc2[0, :N], acc2[1, :N], g2, dinv, b2[None, :],
                   batch2d, Wl, bl[None, :])
    return out
